# baseline (speedup 1.0000x reference)
import functools

import jax
import jax.numpy as jnp
from jax import lax
from jax.experimental import pallas as pl
from jax.experimental.pallas import tpu as pltpu
from jax.experimental.pallas import tpu_sc as plsc

EPS = 1e-5

NC, NS = 2, 16
NW = NC * NS
GRP = 128


def _normalize_table(table, w, b):
    V, D = table.shape
    RB = 2000
    assert V % RB == 0

    def body(t_ref, w_ref, b_ref, o_ref):
        e = t_ref[...]
        mu = jnp.mean(e, axis=-1, keepdims=True)
        d = e - mu
        var = jnp.mean(d * d, axis=-1, keepdims=True)
        o_ref[...] = d * lax.rsqrt(var + EPS) * w_ref[...] + b_ref[...]

    return pl.pallas_call(
        body,
        grid=(V // RB,),
        in_specs=[
            pl.BlockSpec((RB, D), lambda i: (i, 0)),
            pl.BlockSpec((1, D), lambda i: (0, 0)),
            pl.BlockSpec((1, D), lambda i: (0, 0)),
        ],
        out_specs=pl.BlockSpec((RB, D), lambda i: (i, 0)),
        out_shape=jax.ShapeDtypeStruct((V, D), jnp.float32),
    )(table, w.reshape(1, D), b.reshape(1, D))


def _gather_mask_sc(ntable, x):
    V, D = ntable.shape
    B, S = x.shape
    rows_per_w = B // NW
    assert rows_per_w * NW == B

    mesh = plsc.VectorSubcoreMesh(
        core_axis_name="c", subcore_axis_name="s",
        num_cores=NC, num_subcores=NS)

    mgroups = []
    g = 0
    while g * 16 + 16 <= S:
        mgroups.append(g * 16)
        g += 1
    if mgroups[-1] + 16 < S:
        mgroups.append(S - 16)

    gchunks = []
    off = 0
    while off < S:
        n = min(GRP, S - off)
        gchunks.append((off, n))
        off += n

    @functools.partial(
        pl.kernel,
        out_type=[
            jax.ShapeDtypeStruct((B, S, D), jnp.float32),
            jax.ShapeDtypeStruct((B * S,), jnp.int32),
        ],
        mesh=mesh,
        compiler_params=pltpu.CompilerParams(use_tc_tiling_on_sc=False),
        scratch_types=[
            pltpu.VMEM((S,), jnp.int32),
            pltpu.VMEM((S, D), jnp.float32),
            pltpu.VMEM((rows_per_w * S,), jnp.int32),
            pltpu.SemaphoreType.DMA,
        ],
    )
    def k(tab_hbm, x_hbm, out_hbm, mask_hbm, idx_v, rows_v, mask_v, sem):
        wid = lax.axis_index("s") * NC + lax.axis_index("c")
        xr0 = wid * rows_per_w

        def block(i, carry):
            xr = xr0 + i
            pltpu.sync_copy(x_hbm.at[xr], idx_v)
            cps = [
                pltpu.async_copy(
                    tab_hbm.at[idx_v.at[pl.ds(off, n)]],
                    rows_v.at[pl.ds(off, n)], sem)
                for off, n in gchunks
            ]
            for off in mgroups:
                iv = idx_v[pl.ds(off, 16)]
                mask_v[pl.ds(i * S + off, 16)] = jnp.where(
                    iv > 0, jnp.int32(1), jnp.int32(0))
            for cp in cps:
                cp.wait()
            pltpu.sync_copy(rows_v, out_hbm.at[xr])
            return carry

        lax.fori_loop(0, rows_per_w, block, 0)
        pltpu.sync_copy(
            mask_v, mask_hbm.at[pl.ds(xr0 * S, rows_per_w * S)])

    return k(ntable, x)


def kernel(x, table, ln_weight, ln_bias):
    B, S = x.shape
    V, D = table.shape
    assert B % NW == 0

    ntable = _normalize_table(table, ln_weight, ln_bias)
    out, mask_i32 = _gather_mask_sc(ntable, x.astype(jnp.int32))
    mask = (mask_i32 != 0).reshape(B, S)
    return (out, mask)


# speedup vs baseline: 1.2079x; 1.2079x over previous
import functools

import jax
import jax.numpy as jnp
from jax import lax
from jax.experimental import pallas as pl
from jax.experimental.pallas import tpu as pltpu
from jax.experimental.pallas import tpu_sc as plsc

EPS = 1e-5

NC, NS = 2, 16
NW = NC * NS
GRP = 128
KG = 2
BLK = KG * GRP
NBUF = 4


def _normalize_table(table, w, b):
    V, D = table.shape
    RB = 2000
    assert V % RB == 0

    def body(t_ref, w_ref, b_ref, o_ref):
        e = t_ref[...]
        mu = jnp.mean(e, axis=-1, keepdims=True)
        d = e - mu
        var = jnp.mean(d * d, axis=-1, keepdims=True)
        o_ref[...] = d * lax.rsqrt(var + EPS) * w_ref[...] + b_ref[...]

    return pl.pallas_call(
        body,
        grid=(V // RB,),
        in_specs=[
            pl.BlockSpec((RB, D), lambda i: (i, 0)),
            pl.BlockSpec((1, D), lambda i: (0, 0)),
            pl.BlockSpec((1, D), lambda i: (0, 0)),
        ],
        out_specs=pl.BlockSpec((RB, D), lambda i: (i, 0)),
        out_shape=jax.ShapeDtypeStruct((V, D), jnp.float32),
    )(table, w.reshape(1, D), b.reshape(1, D))


def _gather_mask_sc(ntable, x2d):
    V, D = ntable.shape
    NR, _ = x2d.shape
    N = NR * GRP
    per_w = N // NW
    rows_w = per_w // GRP
    nblk = per_w // BLK
    assert per_w * NW == N and nblk * BLK == per_w and nblk % NBUF == 0

    mesh = plsc.VectorSubcoreMesh(
        core_axis_name="c", subcore_axis_name="s",
        num_cores=NC, num_subcores=NS)

    @functools.partial(
        pl.kernel,
        out_type=[
            jax.ShapeDtypeStruct((NR, GRP, D), jnp.float32),
            jax.ShapeDtypeStruct((NR, GRP), jnp.int32),
        ],
        mesh=mesh,
        compiler_params=pltpu.CompilerParams(use_tc_tiling_on_sc=False),
        scratch_types=[
            pltpu.VMEM((rows_w, GRP), jnp.int32),
            pltpu.VMEM((NBUF, KG, GRP, D), jnp.float32),
            pltpu.VMEM((rows_w, GRP), jnp.int32),
        ]
        + [pltpu.SemaphoreType.DMA] * (2 * NBUF),
    )
    def k(tab_hbm, x_hbm, out_hbm, mask_hbm, idx_v, rows_v, mask_v, *sems):
        gat_sems = sems[:NBUF]
        out_sems = sems[NBUF:]
        wid = lax.axis_index("s") * NC + lax.axis_index("c")
        row0 = wid * rows_w

        pltpu.sync_copy(x_hbm.at[pl.ds(row0, rows_w)], idx_v)

        def fire_gathers(g, b):
            for j in range(KG):
                pltpu.async_copy(
                    tab_hbm.at[idx_v.at[g * KG + j]],
                    rows_v.at[b].at[j], gat_sems[b])

        def wait_gathers(b):
            for j in range(KG):
                pltpu.make_async_copy(
                    tab_hbm.at[idx_v.at[j]],
                    rows_v.at[b].at[j], gat_sems[b]).wait()

        def fire_out(g, b):
            pltpu.async_copy(
                rows_v.at[b], out_hbm.at[pl.ds(row0 + g * KG, KG)],
                out_sems[b])

        def wait_out(b):
            pltpu.make_async_copy(
                rows_v.at[b], out_hbm.at[pl.ds(row0, KG)],
                out_sems[b]).wait()

        def mask_chunk(g):
            for j in range(KG):
                for v in range(GRP // 16):
                    iv = idx_v[g * KG + j, pl.ds(v * 16, 16)]
                    mask_v[g * KG + j, pl.ds(v * 16, 16)] = jnp.where(
                        iv > 0, jnp.int32(1), jnp.int32(0))

        def round_body(i, carry):
            for b in range(NBUF):
                g = i * NBUF + b
                @pl.when(i > 0)
                def _(b=b):
                    wait_out(b)
                fire_gathers(g, b)
                pb = b - 1 if b > 0 else NBUF - 1
                pg = g - 1

                @pl.when((i > 0) | (b > 0))
                def _(pb=pb, pg=pg):
                    wait_gathers(pb)
                    fire_out(pg, pb)
                mask_chunk(g)
            return carry

        lax.fori_loop(0, nblk // NBUF, round_body, 0, unroll=False)

        last = nblk - 1
        lb = last % NBUF
        wait_gathers(lb)
        fire_out(last, lb)
        for b in range(NBUF):
            wait_out(b)

        pltpu.sync_copy(mask_v, mask_hbm.at[pl.ds(row0, rows_w)])

    return k(ntable, x2d)


def kernel(x, table, ln_weight, ln_bias):
    B, S = x.shape
    V, D = table.shape
    N = B * S
    assert N % (NW * BLK) == 0

    ntable = _normalize_table(table, ln_weight, ln_bias)
    x2d = x.astype(jnp.int32).reshape(N // GRP, GRP)
    out, mask_i32 = _gather_mask_sc(ntable, x2d)
    normed = out.reshape(B, S, D)
    mask = (mask_i32 != 0).reshape(B, S)
    return (normed, mask)
